# Initial kernel scaffold; baseline (speedup 1.0000x reference)
import jax
import jax.numpy as jnp
from jax.experimental import pallas as pl


def kernel(t, table, W1, b1, W2, b2):
    raise NotImplementedError("write your pallas kernel here")


# speedup vs baseline: 5.0034x; 5.0034x over previous
import functools

import jax
import jax.numpy as jnp
import numpy as np
from jax import lax
from jax.experimental import pallas as pl
from jax.experimental.pallas import tpu as pltpu
from jax.experimental.pallas import tpu_sc as plsc

_D = 64
_ROW_BLOCK = 1000
_NC = 2
_NS = 16
_NW = _NC * _NS
_CHUNK = 128


def _mlp_body(tbl_ref, w1_ref, b1_ref, w2_ref, b2_ref, out_ref):
    x = tbl_ref[...]
    rows = lax.broadcasted_iota(jnp.int32, x.shape, 0)
    first_block = pl.program_id(0) == 0
    x = jnp.where(jnp.logical_and(first_block, rows == 0), 0.0, x)
    h = lax.dot_general(x, w1_ref[...], (((1,), (1,)), ((), ())),
                        preferred_element_type=jnp.float32)
    h = h + b1_ref[...]
    h = h * 0.5 * (1.0 + lax.erf(h * np.float32(1.0 / np.sqrt(2.0))))
    o = lax.dot_general(h, w2_ref[...], (((1,), (1,)), ((), ())),
                        preferred_element_type=jnp.float32)
    out_ref[...] = o + b2_ref[...]


def _transform_table(table, W1, b1, W2, b2):
    n = table.shape[0]
    return pl.pallas_call(
        _mlp_body,
        grid=(n // _ROW_BLOCK,),
        in_specs=[
            pl.BlockSpec((_ROW_BLOCK, _D), lambda i: (i, 0)),
            pl.BlockSpec((_D, _D), lambda i: (0, 0)),
            pl.BlockSpec((1, _D), lambda i: (0, 0)),
            pl.BlockSpec((_D, _D), lambda i: (0, 0)),
            pl.BlockSpec((1, _D), lambda i: (0, 0)),
        ],
        out_specs=pl.BlockSpec((_ROW_BLOCK, _D), lambda i: (i, 0)),
        out_shape=jax.ShapeDtypeStruct((n, _D), jnp.float32),
    )(table, W1, b1.reshape(1, _D), W2, b2.reshape(1, _D))


def _sc_gather(tbl2, idx3):
    n_chunks = idx3.shape[1]
    b_per_w = n_chunks * _CHUNK
    B = _NW * b_per_w
    mesh = plsc.VectorSubcoreMesh(core_axis_name="c", subcore_axis_name="s")

    @functools.partial(
        pl.kernel, mesh=mesh,
        compiler_params=pltpu.CompilerParams(use_tc_tiling_on_sc=False),
        out_type=jax.ShapeDtypeStruct((B, _D), jnp.float32),
        scratch_types=[
            pltpu.VMEM((n_chunks, _CHUNK), jnp.int32),
            pltpu.VMEM((_CHUNK, _D), jnp.float32),
            pltpu.SemaphoreType.DMA,
        ],
    )
    def k(tbl_hbm, idx_hbm, out_hbm, idx_v, rows_v, sem):
        wid = lax.axis_index("s") * _NC + lax.axis_index("c")
        base = wid * b_per_w
        pltpu.sync_copy(idx_hbm.at[wid], idx_v)

        def body(j, carry):
            pltpu.async_copy(tbl_hbm.at[idx_v.at[j]], rows_v, sem).wait()
            pltpu.sync_copy(rows_v, out_hbm.at[pl.ds(base + j * _CHUNK, _CHUNK)])
            return carry

        lax.fori_loop(0, n_chunks, body, 0)

    return k(tbl2, idx3)


def kernel(t, table, W1, b1, W2, b2):
    tbl2 = _transform_table(table, W1, b1, W2, b2)
    B, L = t.shape
    idx3 = t.astype(jnp.int32).reshape(_NW, (B * L) // (_NW * _CHUNK), _CHUNK)
    flat = _sc_gather(tbl2, idx3)
    return flat.reshape(B, L, _D)


# speedup vs baseline: 5.8518x; 1.1696x over previous
import functools

import jax
import jax.numpy as jnp
import numpy as np
from jax import lax
from jax.experimental import pallas as pl
from jax.experimental.pallas import tpu as pltpu
from jax.experimental.pallas import tpu_sc as plsc

_D = 64
_ROW_BLOCK = 1000
_NC = 2
_NS = 16
_NW = _NC * _NS
_CHUNK = 128
_NBUF = 8
_GAP = 6


def _mlp_body(tbl_ref, w1_ref, b1_ref, w2_ref, b2_ref, out_ref):
    x = tbl_ref[...]
    rows = lax.broadcasted_iota(jnp.int32, x.shape, 0)
    first_block = pl.program_id(0) == 0
    x = jnp.where(jnp.logical_and(first_block, rows == 0), 0.0, x)
    h = lax.dot_general(x, w1_ref[...], (((1,), (1,)), ((), ())),
                        preferred_element_type=jnp.float32)
    h = h + b1_ref[...]
    h = h * 0.5 * (1.0 + lax.erf(h * np.float32(1.0 / np.sqrt(2.0))))
    o = lax.dot_general(h, w2_ref[...], (((1,), (1,)), ((), ())),
                        preferred_element_type=jnp.float32)
    out_ref[...] = o + b2_ref[...]


def _transform_table(table, W1, b1, W2, b2):
    n = table.shape[0]
    return pl.pallas_call(
        _mlp_body,
        grid=(n // _ROW_BLOCK,),
        in_specs=[
            pl.BlockSpec((_ROW_BLOCK, _D), lambda i: (i, 0)),
            pl.BlockSpec((_D, _D), lambda i: (0, 0)),
            pl.BlockSpec((1, _D), lambda i: (0, 0)),
            pl.BlockSpec((_D, _D), lambda i: (0, 0)),
            pl.BlockSpec((1, _D), lambda i: (0, 0)),
        ],
        out_specs=pl.BlockSpec((_ROW_BLOCK, _D), lambda i: (i, 0)),
        out_shape=jax.ShapeDtypeStruct((n, _D), jnp.float32),
    )(table, W1, b1.reshape(1, _D), W2, b2.reshape(1, _D))


def _sc_gather(tbl2, idx3):
    n_chunks = idx3.shape[1]
    b_per_w = n_chunks * _CHUNK
    B = _NW * b_per_w
    mesh = plsc.VectorSubcoreMesh(core_axis_name="c", subcore_axis_name="s")

    @functools.partial(
        pl.kernel, mesh=mesh,
        compiler_params=pltpu.CompilerParams(use_tc_tiling_on_sc=False),
        out_type=jax.ShapeDtypeStruct((B, _D), jnp.float32),
        scratch_types=[
            pltpu.VMEM((n_chunks, _CHUNK), jnp.int32),
            pltpu.VMEM((_NBUF, _CHUNK, _D), jnp.float32),
            pltpu.SemaphoreType.DMA((_NBUF,)),
            pltpu.SemaphoreType.DMA((_NBUF,)),
        ],
    )
    def k(tbl_hbm, idx_hbm, out_hbm, idx_v, rows_v, gsem, ssem):
        wid = lax.axis_index("s") * _NC + lax.axis_index("c")
        base = wid * b_per_w
        pltpu.sync_copy(idx_hbm.at[wid], idx_v)

        def fire_gather(j, slot):
            pltpu.async_copy(tbl_hbm.at[idx_v.at[j]], rows_v.at[slot],
                             gsem.at[slot])

        def wait_gather(j, slot):
            pltpu.make_async_copy(tbl_hbm.at[idx_v.at[j]], rows_v.at[slot],
                                  gsem.at[slot]).wait()

        def fire_scatter(j, slot):
            pltpu.async_copy(rows_v.at[slot],
                             out_hbm.at[pl.ds(base + j * _CHUNK, _CHUNK)],
                             ssem.at[slot])

        def wait_scatter(j, slot):
            pltpu.make_async_copy(rows_v.at[slot],
                                  out_hbm.at[pl.ds(base + j * _CHUNK, _CHUNK)],
                                  ssem.at[slot]).wait()

        for b in range(_NBUF):
            fire_gather(b, b)
            if b >= _GAP:
                wait_gather(b - _GAP, b - _GAP)
                fire_scatter(b - _GAP, b - _GAP)

        def body(t, carry):
            for b in range(_NBUF):
                j = t * _NBUF + b
                slot_s = (b + _NBUF - _GAP) % _NBUF
                wait_scatter(j - _NBUF, b)
                fire_gather(j, b)
                wait_gather(j - _GAP, slot_s)
                fire_scatter(j - _GAP, slot_s)
            return carry

        lax.fori_loop(1, n_chunks // _NBUF, body, 0)

        for jj in range(n_chunks - _GAP, n_chunks):
            wait_gather(jj, jj % _NBUF)
            fire_scatter(jj, jj % _NBUF)
        for jj in range(n_chunks - _NBUF, n_chunks):
            wait_scatter(jj, jj % _NBUF)

    return k(tbl2, idx3)


def kernel(t, table, W1, b1, W2, b2):
    tbl2 = _transform_table(table, W1, b1, W2, b2)
    B, L = t.shape
    idx3 = t.astype(jnp.int32).reshape(_NW, (B * L) // (_NW * _CHUNK), _CHUNK)
    flat = _sc_gather(tbl2, idx3)
    return flat.reshape(B, L, _D)
